# baseline (speedup 1.0000x reference)
import jax
import jax.numpy as jnp
from jax.experimental import pallas as pl

_G = 76
_CELLS = _G * _G
_NA = 3
_ATTRS = 85
_STRIDE = 8.0
_ANCH_W = (10.0, 16.0, 33.0)
_ANCH_H = (13.0, 30.0, 23.0)


def _dl_kernel(x_ref, o_ref):
    v = x_ref[0].reshape(_NA * _ATTRS, _CELLS)
    r = jax.lax.broadcasted_iota(jnp.int32, v.shape, 0)
    j = jax.lax.broadcasted_iota(jnp.int32, v.shape, 1)
    colf = (j % _G).astype(jnp.float32)
    rowf = (j // _G).astype(jnp.float32)
    a = r // _ATTRS
    attr = r - a * _ATTRS
    aw = jnp.where(a == 0, _ANCH_W[0], jnp.where(a == 1, _ANCH_W[1], _ANCH_W[2]))
    ah = jnp.where(a == 0, _ANCH_H[0], jnp.where(a == 1, _ANCH_H[1], _ANCH_H[2]))
    is_w = attr == 2
    is_wh = is_w | (attr == 3)
    e = jnp.exp(jnp.where(is_wh, v, -v))
    val = jnp.where(is_wh, e * jnp.where(is_w, aw, ah), 1.0 / (1.0 + e))
    off = jnp.where(attr == 0, colf, jnp.where(attr == 1, rowf, 0.0))
    scale = jnp.where(attr < 2, _STRIDE, 1.0)
    val = (val + off) * scale
    o_ref[0] = val.T


def kernel(x):
    b = x.shape[0]
    out = pl.pallas_call(
        _dl_kernel,
        grid=(b,),
        in_specs=[pl.BlockSpec((1, _NA * _ATTRS, _G, _G), lambda bi: (bi, 0, 0, 0))],
        out_specs=pl.BlockSpec((1, _CELLS, _NA * _ATTRS), lambda bi: (bi, 0, 0)),
        out_shape=jax.ShapeDtypeStruct((b, _CELLS, _NA * _ATTRS), jnp.float32),
    )(x)
    return out.reshape(b, _CELLS * _NA, _ATTRS)


# speedup vs baseline: 1.1054x; 1.1054x over previous
import jax
import jax.numpy as jnp
from jax.experimental import pallas as pl
from jax.experimental.pallas import tpu as pltpu

_G = 76
_CELLS = _G * _G
_NA = 3
_ATTRS = 85
_STRIDE = 8.0
_ANCH_W = (10.0, 16.0, 33.0)
_ANCH_H = (13.0, 30.0, 23.0)


def _dl_kernel(x_ref, o_ref):
    v = x_ref[0]
    r = jax.lax.broadcasted_iota(jnp.int32, v.shape, 0)
    j = jax.lax.broadcasted_iota(jnp.int32, v.shape, 1)
    colf = (j % _G).astype(jnp.float32)
    rowf = (j // _G).astype(jnp.float32)
    a = r // _ATTRS
    attr = r - a * _ATTRS
    aw = jnp.where(a == 0, _ANCH_W[0], jnp.where(a == 1, _ANCH_W[1], _ANCH_W[2]))
    ah = jnp.where(a == 0, _ANCH_H[0], jnp.where(a == 1, _ANCH_H[1], _ANCH_H[2]))
    is_w = attr == 2
    is_wh = is_w | (attr == 3)
    e = jnp.exp(jnp.where(is_wh, v, -v))
    val = jnp.where(is_wh, e * jnp.where(is_w, aw, ah), 1.0 / (1.0 + e))
    off = jnp.where(attr == 0, colf, jnp.where(attr == 1, rowf, 0.0))
    scale = jnp.where(attr < 2, _STRIDE, 1.0)
    val = (val + off) * scale
    o_ref[0] = val.T


def kernel(x):
    b = x.shape[0]
    xf = x.reshape(b, _NA * _ATTRS, _CELLS)
    out = pl.pallas_call(
        _dl_kernel,
        grid=(b,),
        in_specs=[pl.BlockSpec((1, _NA * _ATTRS, _CELLS), lambda bi: (bi, 0, 0))],
        out_specs=pl.BlockSpec((1, _CELLS, _NA * _ATTRS), lambda bi: (bi, 0, 0)),
        out_shape=jax.ShapeDtypeStruct((b, _CELLS, _NA * _ATTRS), jnp.float32),
        compiler_params=pltpu.CompilerParams(
            dimension_semantics=("parallel",),
        ),
    )(xf)
    return out.reshape(b, _CELLS * _NA, _ATTRS)
